# baseline (speedup 1.0000x reference)
import functools

import jax
import jax.numpy as jnp
from jax import lax
from jax.experimental import pallas as pl
from jax.experimental.pallas import tpu as pltpu
from jax.experimental.pallas import tpu_sc as plsc

D = 64
EPS = 1e-5

_info = plsc.get_sparse_core_info()
NC, NS = _info.num_cores, _info.num_subcores
NW = NC * NS

CHUNK = 256
IDXW = 128
GPC = CHUNK // IDXW
STAGE = 1024
CPS = STAGE // CHUNK
SROWS = STAGE // IDXW

BB = 256


def _make_gather(n_rows):
    assert n_rows % (NW * STAGE) == 0
    rows_per_w = n_rows // NW
    n_groups = rows_per_w // STAGE
    mesh = plsc.VectorSubcoreMesh(core_axis_name="c", subcore_axis_name="s")

    @functools.partial(
        pl.kernel,
        mesh=mesh,
        compiler_params=pltpu.CompilerParams(use_tc_tiling_on_sc=False),
        out_type=jax.ShapeDtypeStruct((n_rows, D), jnp.float32),
        scratch_types=[
            pltpu.VMEM((SROWS, IDXW), jnp.int32),
            pltpu.VMEM((CHUNK, D), jnp.float32),
            pltpu.SemaphoreType.DMA,
        ],
    )
    def gather_k(x_hbm, w_hbm, out_hbm, idx_v, rows_v, sem):
        wid = lax.axis_index("s") * NC + lax.axis_index("c")
        base = wid * rows_per_w

        def group_body(g, _):
            grow0 = base + g * STAGE
            goff = pl.multiple_of(grow0 // IDXW, 8)
            pltpu.sync_copy(x_hbm.at[pl.ds(goff, SROWS)], idx_v)
            for c in range(CPS):
                row0 = grow0 + c * CHUNK
                for j in range(GPC):
                    pltpu.async_copy(
                        w_hbm.at[idx_v.at[c * GPC + j]],
                        rows_v.at[pl.ds(j * IDXW, IDXW)], sem).wait()
                pltpu.sync_copy(rows_v, out_hbm.at[pl.ds(row0, CHUNK)])
            return 0

        lax.fori_loop(0, n_groups, group_body, 0)

    return gather_k


def _ln_tc(mid_ref, m_ref, lnw2_ref, lnb2_ref, out_ref):
    x = mid_ref[...]
    m = m_ref[...]
    mean = jax.lax.dot(x, m, precision=jax.lax.Precision.HIGHEST)
    ex2 = jax.lax.dot(x * x, m, precision=jax.lax.Precision.HIGHEST)
    rstd = lax.rsqrt(ex2 - mean * mean + EPS)
    normed = (x - mean) * rstd * lnw2_ref[...] + lnb2_ref[...]
    pair = jnp.concatenate(
        [normed[:, None, :D], normed[:, None, D:]], axis=1)
    out_ref[...] = pair.reshape(BB, 25, 2, D).reshape(BB, 50, D)


def _make_ln(n_rows, h):
    n_lines = n_rows // 2
    lines_pb = n_lines // (n_rows // h)
    nb = n_rows // h
    grid = nb // BB

    return pl.pallas_call(
        _ln_tc,
        grid=(grid,),
        in_specs=[
            pl.BlockSpec((BB * lines_pb, 128), lambda i: (i, 0)),
            pl.BlockSpec((128, 128), lambda i: (0, 0)),
            pl.BlockSpec((2 * D,), lambda i: (0,)),
            pl.BlockSpec((2 * D,), lambda i: (0,)),
        ],
        out_specs=pl.BlockSpec((BB, h, D), lambda i: (i, 0, 0)),
        out_shape=jax.ShapeDtypeStruct((nb, h, D), jnp.float32),
    )


def kernel(x, weight, ln_weight, ln_bias):
    b, h = x.shape
    n = b * h
    x2 = x.reshape(n // IDXW, IDXW).astype(jnp.int32)
    mid = _make_gather(n)(x2, weight)
    mid2 = mid.reshape(n // 2, 128)
    eye2 = jnp.kron(jnp.eye(2, dtype=jnp.float32),
                    jnp.full((D, D), 1.0 / D, jnp.float32))
    lnw2 = jnp.concatenate([ln_weight, ln_weight])
    lnb2 = jnp.concatenate([ln_bias, ln_bias])
    return _make_ln(n, h)(mid2, eye2, lnw2, lnb2)


# speedup vs baseline: 1.0328x; 1.0328x over previous
import functools

import jax
import jax.numpy as jnp
from jax import lax
from jax.experimental import pallas as pl
from jax.experimental.pallas import tpu as pltpu
from jax.experimental.pallas import tpu_sc as plsc

D = 64
EPS = 1e-5

_info = plsc.get_sparse_core_info()
NC, NS = _info.num_cores, _info.num_subcores
NW = NC * NS

CHUNK = 256
IDXW = 128
GPC = CHUNK // IDXW
STAGE = 1024
CPS = STAGE // CHUNK
SROWS = STAGE // IDXW

BB = 256


def _make_gather(n_rows):
    assert n_rows % (NW * STAGE) == 0
    rows_per_w = n_rows // NW
    n_groups = rows_per_w // STAGE
    mesh = plsc.VectorSubcoreMesh(core_axis_name="c", subcore_axis_name="s")

    @functools.partial(
        pl.kernel,
        mesh=mesh,
        compiler_params=pltpu.CompilerParams(use_tc_tiling_on_sc=False),
        out_type=jax.ShapeDtypeStruct((n_rows, D), jnp.float32),
        scratch_types=[
            pltpu.VMEM((SROWS, IDXW), jnp.int32),
            pltpu.VMEM((CHUNK, D), jnp.float32),
            pltpu.SemaphoreType.DMA,
        ],
    )
    def gather_k(x_hbm, w_hbm, out_hbm, idx_v, rows_v, sem):
        wid = lax.axis_index("s") * NC + lax.axis_index("c")
        base = wid * rows_per_w

        def group_body(g, _):
            grow0 = base + g * STAGE
            goff = pl.multiple_of(grow0 // IDXW, 8)
            pltpu.sync_copy(x_hbm.at[pl.ds(goff, SROWS)], idx_v)
            for c in range(CPS):
                row0 = grow0 + c * CHUNK
                for j in range(GPC):
                    pltpu.async_copy(
                        w_hbm.at[idx_v.at[c * GPC + j]],
                        rows_v.at[pl.ds(j * IDXW, IDXW)], sem).wait()
                pltpu.sync_copy(rows_v, out_hbm.at[pl.ds(row0, CHUNK)])
            return 0

        lax.fori_loop(0, n_groups, group_body, 0)

    return gather_k


def _ln_tc(mid_ref, m_ref, lnw2_ref, lnb2_ref, out_ref):
    x = mid_ref[...]
    m = m_ref[...]
    xb = x.astype(jnp.bfloat16)
    mean = jax.lax.dot(xb, m, preferred_element_type=jnp.float32)
    ex2 = jax.lax.dot(xb * xb, m, preferred_element_type=jnp.float32)
    rstd = lax.rsqrt(ex2 - mean * mean + EPS)
    normed = (x - mean) * rstd * lnw2_ref[...] + lnb2_ref[...]
    pair = jnp.concatenate(
        [normed[:, None, :D], normed[:, None, D:]], axis=1)
    out_ref[...] = pair.reshape(BB * 50, D)


def _make_ln(n_rows, h):
    n_lines = n_rows // 2
    lines_pb = n_lines // (n_rows // h)
    nb = n_rows // h
    grid = nb // BB

    return pl.pallas_call(
        _ln_tc,
        grid=(grid,),
        in_specs=[
            pl.BlockSpec((BB * lines_pb, 128), lambda i: (i, 0)),
            pl.BlockSpec((128, 128), lambda i: (0, 0)),
            pl.BlockSpec((2 * D,), lambda i: (0,)),
            pl.BlockSpec((2 * D,), lambda i: (0,)),
        ],
        out_specs=pl.BlockSpec((BB * h, D), lambda i: (i, 0)),
        out_shape=jax.ShapeDtypeStruct((nb * h, D), jnp.float32),
    )


def kernel(x, weight, ln_weight, ln_bias):
    b, h = x.shape
    n = b * h
    x2 = x.reshape(n // IDXW, IDXW).astype(jnp.int32)
    mid = _make_gather(n)(x2, weight)
    mid2 = mid.reshape(n // 2, 128)
    eye2 = jnp.kron(jnp.eye(2, dtype=jnp.bfloat16),
                    jnp.full((D, D), 1.0 / D, jnp.bfloat16))
    lnw2 = jnp.concatenate([ln_weight, ln_weight])
    lnb2 = jnp.concatenate([ln_bias, ln_bias])
    return _make_ln(n, h)(mid2, eye2, lnw2, lnb2).reshape(b, h, D)


# speedup vs baseline: 1.1031x; 1.0680x over previous
import functools

import jax
import jax.numpy as jnp
from jax import lax
from jax.experimental import pallas as pl
from jax.experimental.pallas import tpu as pltpu
from jax.experimental.pallas import tpu_sc as plsc

D = 64
L16 = 16
NV = D // L16
EPS = 1e-5

_info = plsc.get_sparse_core_info()
NC, NS = _info.num_cores, _info.num_subcores
NW = NC * NS

BG = 8


def _rsqrt_nr(x):
    i = lax.bitcast_convert_type(x, jnp.int32)
    i = jnp.int32(0x5F3759DF) - (i >> 1)
    y = lax.bitcast_convert_type(i, jnp.float32)
    for _ in range(3):
        y = y * (1.5 - 0.5 * x * y * y)
    return y


_GDN = lax.GatherDimensionNumbers(
    offset_dims=(), collapsed_slice_dims=(0,), start_index_map=(0,))


def _lane_allsum(v, perms):
    for p in perms:
        pv = lax.gather(v, p, _GDN, slice_sizes=(1,),
                        mode=lax.GatherScatterMode.PROMISE_IN_BOUNDS)
        v = v + pv
    return v


def _make_kernel(nb, h):
    assert nb % (NW * BG) == 0
    b_per_w = nb // NW
    n_groups = b_per_w // BG
    rows_pg = BG * h
    mesh = plsc.VectorSubcoreMesh(core_axis_name="c", subcore_axis_name="s")

    @functools.partial(
        pl.kernel,
        mesh=mesh,
        out_type=jax.ShapeDtypeStruct((nb, h, D), jnp.float32),
        scratch_types=[
            pltpu.VMEM((BG, h), jnp.int32),
            pltpu.VMEM((rows_pg, 2 * D), jnp.float32),
            pltpu.VMEM((BG, h, D), jnp.float32),
            pltpu.VMEM((D,), jnp.float32),
            pltpu.VMEM((D,), jnp.float32),
            pltpu.SemaphoreType.DMA,
        ],
    )
    def emb_ln(x_hbm, w_hbm, lnw_hbm, lnb_hbm, out_hbm,
               idx_v, rows_v, out_v, lnw_v, lnb_v, sem):
        wid = lax.axis_index("s") * NC + lax.axis_index("c")
        base = wid * b_per_w

        pltpu.sync_copy(lnw_hbm, lnw_v)
        pltpu.sync_copy(lnb_hbm, lnb_v)
        w_vecs = [lnw_v[pl.ds(k * L16, L16)] for k in range(NV)]
        b_vecs = [lnb_v[pl.ds(k * L16, L16)] for k in range(NV)]
        lane = lax.iota(jnp.int32, L16)
        perms = [(lane ^ (1 << b))[:, None] for b in range(4)]

        def group_body(g, _):
            b0 = pl.multiple_of(base + g * BG, BG)
            pltpu.sync_copy(x_hbm.at[pl.ds(b0, BG)], idx_v)
            for j in range(BG):
                pltpu.async_copy(
                    w_hbm.at[idx_v.at[j]],
                    rows_v.at[pl.ds(j * h, h)], sem).wait()

            @plsc.parallel_loop(0, rows_pg, unroll=8)
            def row_body(r):
                vs = [rows_v[r, pl.ds(k * L16, L16)] for k in range(NV)]
                s = vs[0] + vs[1] + vs[2] + vs[3]
                q = (vs[0] * vs[0] + vs[1] * vs[1]
                     + vs[2] * vs[2] + vs[3] * vs[3])
                mean = _lane_allsum(s, perms) * (1.0 / D)
                ex2 = _lane_allsum(q, perms) * (1.0 / D)
                rstd = _rsqrt_nr(ex2 - mean * mean + EPS)
                for k in range(NV):
                    out_v[r // h, r % h, pl.ds(k * L16, L16)] = (
                        (vs[k] - mean) * rstd * w_vecs[k] + b_vecs[k])

            pltpu.sync_copy(out_v, out_hbm.at[pl.ds(b0, BG)])
            return 0

        lax.fori_loop(0, n_groups, group_body, 0)

    return emb_ln


def kernel(x, weight, ln_weight, ln_bias):
    b, h = x.shape
    v, d = weight.shape
    wpad = jnp.pad(weight, ((0, 0), (0, 2 * D - d)))
    return _make_kernel(b, h)(x.astype(jnp.int32), wpad, ln_weight, ln_bias)


# speedup vs baseline: 1.4177x; 1.2852x over previous
import functools

import jax
import jax.numpy as jnp
from jax import lax
from jax.experimental import pallas as pl
from jax.experimental.pallas import tpu as pltpu
from jax.experimental.pallas import tpu_sc as plsc

D = 64
L16 = 16
NV = D // L16
EPS = 1e-5

_info = plsc.get_sparse_core_info()
NC, NS = _info.num_cores, _info.num_subcores
NW = NC * NS

BG = 4


def _rsqrt_nr(x):
    i = lax.bitcast_convert_type(x, jnp.int32)
    i = jnp.int32(0x5F3759DF) - (i >> 1)
    y = lax.bitcast_convert_type(i, jnp.float32)
    for _ in range(3):
        y = y * (1.5 - 0.5 * x * y * y)
    return y


_GDN = lax.GatherDimensionNumbers(
    offset_dims=(), collapsed_slice_dims=(0,), start_index_map=(0,))


def _lane_allsum(v, perms):
    for p in perms:
        pv = lax.gather(v, p, _GDN, slice_sizes=(1,),
                        mode=lax.GatherScatterMode.PROMISE_IN_BOUNDS)
        v = v + pv
    return v


def _make_kernel(nb, h):
    assert nb % (NW * BG) == 0
    b_per_w = nb // NW
    n_groups = b_per_w // BG
    rows_pg = BG * h
    mesh = plsc.VectorSubcoreMesh(core_axis_name="c", subcore_axis_name="s")

    @functools.partial(
        pl.kernel,
        mesh=mesh,
        out_type=jax.ShapeDtypeStruct((nb, h, D), jnp.float32),
        scratch_types=[
            pltpu.VMEM((2, BG, h), jnp.int32),
            pltpu.VMEM((2, rows_pg, 2 * D), jnp.float32),
            pltpu.VMEM((BG, h, D), jnp.float32),
            pltpu.VMEM((D,), jnp.float32),
            pltpu.VMEM((D,), jnp.float32),
            pltpu.SemaphoreType.DMA,
            pltpu.SemaphoreType.DMA,
        ],
    )
    def emb_ln(x_hbm, w_hbm, lnw_hbm, lnb_hbm, out_hbm,
               idx_v, rows_v, out_v, lnw_v, lnb_v, sem0, sem1):
        wid = lax.axis_index("s") * NC + lax.axis_index("c")
        base = wid * b_per_w
        sems = [sem0, sem1]

        pltpu.sync_copy(lnw_hbm, lnw_v)
        pltpu.sync_copy(lnb_hbm, lnb_v)
        w_vecs = [lnw_v[pl.ds(k * L16, L16)] for k in range(NV)]
        b_vecs = [lnb_v[pl.ds(k * L16, L16)] for k in range(NV)]
        lane = lax.iota(jnp.int32, L16)
        perms = [(lane ^ (1 << b))[:, None] for b in range(4)]

        def fire(g, buf):
            b0 = jnp.minimum(base + g * BG, nb - BG)
            pltpu.sync_copy(x_hbm.at[pl.ds(b0, BG)], idx_v.at[buf])
            for j in range(BG):
                pltpu.async_copy(
                    w_hbm.at[idx_v.at[buf, j]],
                    rows_v.at[buf, pl.ds(j * h, h)], sems[buf])

        def process(g, buf):
            pltpu.make_async_copy(
                w_hbm.at[pl.ds(0, rows_pg)], rows_v.at[buf],
                sems[buf]).wait()

            @plsc.parallel_loop(0, rows_pg, unroll=8)
            def row_body(r):
                vs = [rows_v[buf, r, pl.ds(k * L16, L16)]
                      for k in range(NV)]
                s = vs[0] + vs[1] + vs[2] + vs[3]
                q = (vs[0] * vs[0] + vs[1] * vs[1]
                     + vs[2] * vs[2] + vs[3] * vs[3])
                mean = _lane_allsum(s, perms) * (1.0 / D)
                ex2 = _lane_allsum(q, perms) * (1.0 / D)
                rstd = _rsqrt_nr(ex2 - mean * mean + EPS)
                for k in range(NV):
                    out_v[r // h, r % h, pl.ds(k * L16, L16)] = (
                        (vs[k] - mean) * rstd * w_vecs[k] + b_vecs[k])

            b0 = pl.multiple_of(base + g * BG, BG)
            pltpu.sync_copy(out_v, out_hbm.at[pl.ds(b0, BG)])

        fire(0, 0)

        def pair_body(k, _):
            g0 = 2 * k
            fire(g0 + 1, 1)
            process(g0, 0)
            fire(g0 + 2, 0)
            process(g0 + 1, 1)
            return 0

        lax.fori_loop(0, n_groups // 2, pair_body, 0)
        pltpu.make_async_copy(
            w_hbm.at[pl.ds(0, rows_pg)], rows_v.at[0], sems[0]).wait()

    return emb_ln


def kernel(x, weight, ln_weight, ln_bias):
    b, h = x.shape
    v, d = weight.shape
    wpad = jnp.pad(weight, ((0, 0), (0, 2 * D - d)))
    return _make_kernel(b, h)(x.astype(jnp.int32), wpad, ln_weight, ln_bias)


# speedup vs baseline: 1.7371x; 1.2253x over previous
import functools

import jax
import jax.numpy as jnp
from jax import lax
from jax.experimental import pallas as pl
from jax.experimental.pallas import tpu as pltpu
from jax.experimental.pallas import tpu_sc as plsc

D = 64
L16 = 16
NV = D // L16
EPS = 1e-5

_info = plsc.get_sparse_core_info()
NC, NS = _info.num_cores, _info.num_subcores
NW = NC * NS

BG = 4


def _rsqrt_nr(x):
    i = lax.bitcast_convert_type(x, jnp.int32)
    i = jnp.int32(0x5F3759DF) - (i >> 1)
    y = lax.bitcast_convert_type(i, jnp.float32)
    for _ in range(3):
        y = y * (1.5 - 0.5 * x * y * y)
    return y


_GDN = lax.GatherDimensionNumbers(
    offset_dims=(), collapsed_slice_dims=(0,), start_index_map=(0,))


def _lane_allsum(v, perms):
    for p in perms:
        pv = lax.gather(v, p, _GDN, slice_sizes=(1,),
                        mode=lax.GatherScatterMode.PROMISE_IN_BOUNDS)
        v = v + pv
    return v


def _make_kernel(nb, h):
    assert nb % (NW * BG) == 0
    b_per_w = nb // NW
    n_groups = b_per_w // BG
    rows_pg = BG * h
    mesh = plsc.VectorSubcoreMesh(core_axis_name="c", subcore_axis_name="s")

    @functools.partial(
        pl.kernel,
        mesh=mesh,
        out_type=jax.ShapeDtypeStruct((nb, h, D), jnp.float32),
        scratch_types=[
            pltpu.VMEM((2, BG, h), jnp.int32),
            pltpu.VMEM((2, rows_pg, 2 * D), jnp.float32),
            pltpu.VMEM((BG, h, D), jnp.float32),
            pltpu.VMEM((D,), jnp.float32),
            pltpu.VMEM((D,), jnp.float32),
            pltpu.SemaphoreType.DMA,
            pltpu.SemaphoreType.DMA,
        ],
    )
    def emb_ln(x_hbm, w_hbm, lnw_hbm, lnb_hbm, out_hbm,
               idx_v, rows_v, out_v, lnw_v, lnb_v, sem0, sem1):
        wid = lax.axis_index("s") * NC + lax.axis_index("c")
        base = wid * b_per_w
        sems = [sem0, sem1]

        pltpu.sync_copy(lnw_hbm, lnw_v)
        pltpu.sync_copy(lnb_hbm, lnb_v)
        w_vecs = [lnw_v[pl.ds(k * L16, L16)] for k in range(NV)]
        b_vecs = [lnb_v[pl.ds(k * L16, L16)] for k in range(NV)]
        lane = lax.iota(jnp.int32, L16)
        perms = [(lane ^ (1 << b))[:, None] for b in range(4)]

        def fire(g, buf):
            b0 = jnp.minimum(base + g * BG, nb - BG)
            pltpu.sync_copy(x_hbm.at[pl.ds(b0, BG)], idx_v.at[buf])
            for j in range(BG):
                pltpu.async_copy(
                    w_hbm.at[idx_v.at[buf, j]],
                    rows_v.at[buf, pl.ds(j * h, h)], sems[buf])

        def process(g, buf):
            pltpu.make_async_copy(
                w_hbm.at[pl.ds(0, rows_pg)], rows_v.at[buf],
                sems[buf]).wait()

            @plsc.parallel_loop(0, rows_pg, unroll=8)
            def row_body(r):
                vs = [rows_v[buf, r, pl.ds(k * L16, L16)]
                      for k in range(NV)]
                s = vs[0] + vs[1] + vs[2] + vs[3]
                q = (vs[0] * vs[0] + vs[1] * vs[1]
                     + vs[2] * vs[2] + vs[3] * vs[3])
                mean = _lane_allsum(s, perms) * (1.0 / D)
                ex2 = _lane_allsum(q, perms) * (1.0 / D)
                rstd = _rsqrt_nr(ex2 - mean * mean + EPS)
                for k in range(NV):
                    out_v[r // h, r % h, pl.ds(k * L16, L16)] = (
                        (vs[k] - mean) * rstd * w_vecs[k] + b_vecs[k])

            b0 = pl.multiple_of(base + g * BG, BG)
            pltpu.sync_copy(out_v, out_hbm.at[pl.ds(b0, BG)])

        fire(0, 0)

        def pair_body(k, _):
            g0 = 2 * k
            fire(g0 + 1, 1)
            process(g0, 0)
            fire(g0 + 2, 0)
            process(g0 + 1, 1)
            return 0

        lax.fori_loop(0, n_groups // 2, pair_body, 0)
        pltpu.make_async_copy(
            w_hbm.at[pl.ds(0, rows_pg)], rows_v.at[0], sems[0]).wait()

    return emb_ln


VB = 7936


def _tpad_tc(wt_ref, out_ref):
    t = wt_ref[...].T
    out_ref[...] = jnp.concatenate(
        [t, jnp.zeros((VB, D), jnp.float32)], axis=1)


def _make_tpad(v):
    return pl.pallas_call(
        _tpad_tc,
        grid=((v + VB - 1) // VB,),
        in_specs=[pl.BlockSpec((D, VB), lambda i: (0, i))],
        out_specs=pl.BlockSpec((VB, 2 * D), lambda i: (i, 0)),
        out_shape=jax.ShapeDtypeStruct((v, 2 * D), jnp.float32),
    )


def kernel(x, weight, ln_weight, ln_bias):
    b, h = x.shape
    v, d = weight.shape
    wpad = _make_tpad(v)(weight.T)
    return _make_kernel(b, h)(x.astype(jnp.int32), wpad, ln_weight, ln_bias)
